# baseline (speedup 1.0000x reference)
import jax
import jax.numpy as jnp
from jax.experimental import pallas as pl
from jax.experimental.pallas import tpu as pltpu

_K = 30
_KK = _K + 1
_LANES = 128
_NEG = -3e38


def _embed_kernel(f_ref, w1_ref, w2_ref, emb_ref):
    h = f_ref[...] * w1_ref[...]
    h = jnp.maximum(h, 0.0)
    h = h * w2_ref[...]
    n = jnp.sqrt(jnp.sum(h * h, axis=1, keepdims=True))
    emb_ref[...] = h / jnp.maximum(n, 1e-12)


def _sim_topk_kernel(q_ref, k_ref, out_ref):
    blk = q_ref.shape[0]
    npad = k_ref.shape[0]
    nout = out_ref.shape[1]
    sim = jax.lax.dot_general(
        q_ref[...], k_ref[...],
        (((1,), (1,)), ((), ())),
        preferred_element_type=jnp.float32,
    )

    a1 = jnp.full((blk, _LANES), _NEG, jnp.float32)
    a2 = a1
    a3 = a1
    a4 = a1
    for j in range(npad // _LANES):
        v = sim[:, j * _LANES:(j + 1) * _LANES]
        h1 = jnp.maximum(a1, v)
        l1 = jnp.minimum(a1, v)
        h2 = jnp.maximum(a2, l1)
        l2 = jnp.minimum(a2, l1)
        h3 = jnp.maximum(a3, l2)
        l3 = jnp.minimum(a3, l2)
        a4 = jnp.maximum(a4, l3)
        a1, a2, a3 = h1, h2, h3

    st = jnp.concatenate([a1.T, a2.T, a3.T, a4.T], axis=0)

    def body(_, m):
        return jnp.max(jnp.where(st < m, st, _NEG), axis=0, keepdims=True)

    t_c = jax.lax.fori_loop(
        0, _K, body, jnp.max(st, axis=0, keepdims=True)).T

    kk = jnp.float32(_KK)
    cnt = jnp.sum((sim >= t_c).astype(jnp.float32), axis=1, keepdims=True)
    hi0 = jnp.max(a1, axis=1, keepdims=True) + 0.1

    def cond(carry):
        it, lo, hi, c = carry
        return jnp.logical_and(it < 40, jnp.any(c != kk))

    def refine(carry):
        it, lo, hi, c = carry
        mid = 0.5 * (lo + hi)
        cm = jnp.sum((sim >= mid).astype(jnp.float32), axis=1, keepdims=True)
        ge = cm >= kk
        lo = jnp.where(ge, mid, lo)
        hi = jnp.where(ge, hi, mid)
        c = jnp.where(ge, cm, c)
        return it + 1, lo, hi, c

    _, t, _, _ = jax.lax.while_loop(cond, refine, (0, t_c, hi0, cnt))

    keep = jnp.where((sim >= t) & (sim > 0.0), sim, 0.0)
    out_ref[...] = jax.lax.slice(keep, (0, 0), (blk, nout))


def kernel(features, w1, w2):
    n, d = features.shape
    w1 = w1.reshape(1, d)
    w2 = w2.reshape(1, d)
    emb = pl.pallas_call(
        _embed_kernel,
        out_shape=jax.ShapeDtypeStruct((n, d), jnp.float32),
    )(features, w1, w2)

    npad = ((n + _LANES - 1) // _LANES) * _LANES
    emb_pad = jnp.pad(emb, ((0, npad - n), (0, 0)))

    blk = 200
    out = pl.pallas_call(
        _sim_topk_kernel,
        grid=(n // blk,),
        in_specs=[
            pl.BlockSpec((blk, d), lambda i: (i, 0)),
            pl.BlockSpec((npad, d), lambda i: (0, 0)),
        ],
        out_specs=pl.BlockSpec((blk, n), lambda i: (i, 0)),
        out_shape=jax.ShapeDtypeStruct((n, n), jnp.float32),
        compiler_params=pltpu.CompilerParams(
            dimension_semantics=("parallel",)),
    )(emb, emb_pad)
    return out


# speedup vs baseline: 1.4035x; 1.4035x over previous
import jax
import jax.numpy as jnp
from jax.experimental import pallas as pl
from jax.experimental.pallas import tpu as pltpu

_K = 30
_KK = _K + 1
_LANES = 128
_RG = 40
_NEG = -3e38


def _embed_kernel(f_ref, w1_ref, w2_ref, emb_ref):
    h = f_ref[...] * w1_ref[...]
    h = jnp.maximum(h, 0.0)
    h = h * w2_ref[...]
    n = jnp.sqrt(jnp.sum(h * h, axis=1, keepdims=True))
    emb_ref[...] = h / jnp.maximum(n, 1e-12)


def _sim_topk_kernel(q_ref, k_ref, out_ref, sim_ref):
    blk = q_ref.shape[0]
    npad = k_ref.shape[0]
    nout = out_ref.shape[1]
    ntiles = npad // _LANES

    sim_ref[...] = jax.lax.dot_general(
        q_ref[...], k_ref[...],
        (((1,), (1,)), ((), ())),
        preferred_element_type=jnp.float32,
    )

    st_cols = []
    a5m_rows = []
    for g in range(blk // _RG):
        r0 = g * _RG
        a1 = jnp.full((_RG, _LANES), _NEG, jnp.float32)
        a2 = a1
        a3 = a1
        a4 = a1
        a5 = a1
        for j in range(ntiles):
            v = sim_ref[r0:r0 + _RG, j * _LANES:(j + 1) * _LANES]
            h1 = jnp.maximum(a1, v)
            l1 = jnp.minimum(a1, v)
            h2 = jnp.maximum(a2, l1)
            l2 = jnp.minimum(a2, l1)
            h3 = jnp.maximum(a3, l2)
            l3 = jnp.minimum(a3, l2)
            h4 = jnp.maximum(a4, l3)
            l4 = jnp.minimum(a4, l3)
            a5 = jnp.maximum(a5, l4)
            a1, a2, a3, a4 = h1, h2, h3, h4
        st_cols.append(
            jnp.concatenate([a1.T, a2.T, a3.T, a4.T], axis=0))
        a5m_rows.append(jnp.max(a5, axis=1, keepdims=True))

    st = jnp.concatenate(st_cols, axis=1)

    def body(_, m):
        return jnp.max(jnp.where(st < m, st, _NEG), axis=0, keepdims=True)

    t_c = jax.lax.fori_loop(
        0, _K, body, jnp.max(st, axis=0, keepdims=True)).T

    a5m = jnp.concatenate(a5m_rows, axis=0)
    kk = jnp.float32(_KK)
    cnt0 = jnp.where(a5m < t_c, kk, 0.0)
    hi0 = jnp.max(st, axis=0, keepdims=True).T + 0.1

    def _count(t):
        acc = jnp.zeros((blk, _LANES), jnp.float32)
        for j in range(ntiles):
            v = sim_ref[:, j * _LANES:(j + 1) * _LANES]
            acc = acc + jnp.where(v >= t, 1.0, 0.0)
        return jnp.sum(acc, axis=1, keepdims=True)

    def cond(carry):
        it, lo, hi, c = carry
        return jnp.logical_and(it < 40, jnp.any(c != kk))

    def refine(carry):
        it, lo, hi, c = carry
        mid = 0.5 * (lo + hi)
        cm = _count(mid)
        ge = cm >= kk
        lo = jnp.where(ge, mid, lo)
        hi = jnp.where(ge, hi, mid)
        c = jnp.where(ge, cm, c)
        return it + 1, lo, hi, c

    _, t, _, _ = jax.lax.while_loop(cond, refine, (0, t_c, hi0, cnt0))

    t_eff = jnp.maximum(t, 1e-37)
    keep = sim_ref[:, :nout]
    out_ref[...] = jnp.where(keep >= t_eff, keep, 0.0)


def kernel(features, w1, w2):
    n, d = features.shape
    w1 = w1.reshape(1, d)
    w2 = w2.reshape(1, d)
    emb = pl.pallas_call(
        _embed_kernel,
        out_shape=jax.ShapeDtypeStruct((n, d), jnp.float32),
    )(features, w1, w2)

    npad = ((n + _LANES - 1) // _LANES) * _LANES
    emb_pad = jnp.pad(emb, ((0, npad - n), (0, 0)))

    blk = 200
    out = pl.pallas_call(
        _sim_topk_kernel,
        grid=(n // blk,),
        in_specs=[
            pl.BlockSpec((blk, d), lambda i: (i, 0)),
            pl.BlockSpec((npad, d), lambda i: (0, 0)),
        ],
        out_specs=pl.BlockSpec((blk, n), lambda i: (i, 0)),
        out_shape=jax.ShapeDtypeStruct((n, n), jnp.float32),
        scratch_shapes=[pltpu.VMEM((blk, npad), jnp.float32)],
        compiler_params=pltpu.CompilerParams(
            dimension_semantics=("arbitrary",)),
    )(emb, emb_pad)
    return out


# speedup vs baseline: 1.4948x; 1.0650x over previous
import jax
import jax.numpy as jnp
from jax.experimental import pallas as pl
from jax.experimental.pallas import tpu as pltpu

_K = 30
_KK = _K + 1
_LANES = 128
_RG = 40
_NEG = -3e38


def _embed_kernel(f_ref, w1_ref, w2_ref, emb_ref):
    h = f_ref[...] * w1_ref[...]
    h = jnp.maximum(h, 0.0)
    h = h * w2_ref[...]
    n = jnp.sqrt(jnp.sum(h * h, axis=1, keepdims=True))
    emb_ref[...] = h / jnp.maximum(n, 1e-12)


def _sim_topk_kernel(q_ref, k_ref, out_ref, sim_ref):
    blk = q_ref.shape[0]
    npad = k_ref.shape[0]
    nout = out_ref.shape[1]
    ntiles = npad // _LANES

    sim_ref[...] = jax.lax.dot_general(
        q_ref[...], k_ref[...],
        (((1,), (1,)), ((), ())),
        preferred_element_type=jnp.float32,
    )

    st_cols = []
    a5m_rows = []
    for g in range(blk // _RG):
        r0 = g * _RG
        a1 = jnp.full((_RG, _LANES), _NEG, jnp.float32)
        a2 = a1
        a3 = a1
        a4 = a1
        a5 = a1
        for j in range(ntiles):
            v = sim_ref[r0:r0 + _RG, j * _LANES:(j + 1) * _LANES]
            h1 = jnp.maximum(a1, v)
            l1 = jnp.minimum(a1, v)
            h2 = jnp.maximum(a2, l1)
            l2 = jnp.minimum(a2, l1)
            h3 = jnp.maximum(a3, l2)
            l3 = jnp.minimum(a3, l2)
            h4 = jnp.maximum(a4, l3)
            l4 = jnp.minimum(a4, l3)
            a5 = jnp.maximum(a5, l4)
            a1, a2, a3, a4 = h1, h2, h3, h4
        st_cols.append(
            jnp.concatenate([a1.T, a2.T, a3.T, a4.T], axis=0))
        a5m_rows.append(jnp.max(a5, axis=1, keepdims=True))

    st = jnp.concatenate(st_cols, axis=1)

    def body(_, m):
        return jnp.max(jnp.where(st < m, st, _NEG), axis=0, keepdims=True)

    t_c = jax.lax.fori_loop(
        0, _K, body, jnp.max(st, axis=0, keepdims=True)).T

    a5m = jnp.concatenate(a5m_rows, axis=0)
    kk = jnp.float32(_KK)
    cnt0 = jnp.where(a5m < t_c, kk, 0.0)
    hi0 = jnp.max(st, axis=0, keepdims=True).T + 0.1

    def _count(t):
        acc = jnp.zeros((blk, _LANES), jnp.float32)
        for j in range(ntiles):
            v = sim_ref[:, j * _LANES:(j + 1) * _LANES]
            acc = acc + jnp.where(v >= t, 1.0, 0.0)
        return jnp.sum(acc, axis=1, keepdims=True)

    def cond(carry):
        it, lo, hi, c = carry
        return jnp.logical_and(it < 40, jnp.any(c != kk))

    def refine(carry):
        it, lo, hi, c = carry
        mid = 0.5 * (lo + hi)
        cm = _count(mid)
        ge = cm >= kk
        lo = jnp.where(ge, mid, lo)
        hi = jnp.where(ge, hi, mid)
        c = jnp.where(ge, cm, c)
        return it + 1, lo, hi, c

    _, t, _, _ = jax.lax.while_loop(cond, refine, (0, t_c, hi0, cnt0))

    t_eff = jnp.maximum(t, 1e-37)
    keep = sim_ref[:, :nout]
    out_ref[...] = jnp.where(keep >= t_eff, keep, 0.0)


def kernel(features, w1, w2):
    n, d = features.shape
    w1 = w1.reshape(1, d)
    w2 = w2.reshape(1, d)
    emb = pl.pallas_call(
        _embed_kernel,
        out_shape=jax.ShapeDtypeStruct((n, d), jnp.float32),
    )(features, w1, w2)

    npad = ((n + _LANES - 1) // _LANES) * _LANES
    emb_pad = jnp.pad(emb, ((0, npad - n), (0, 0)))

    blk = 400
    out = pl.pallas_call(
        _sim_topk_kernel,
        grid=(n // blk,),
        in_specs=[
            pl.BlockSpec((blk, d), lambda i: (i, 0)),
            pl.BlockSpec((npad, d), lambda i: (0, 0)),
        ],
        out_specs=pl.BlockSpec((blk, n), lambda i: (i, 0)),
        out_shape=jax.ShapeDtypeStruct((n, n), jnp.float32),
        scratch_shapes=[pltpu.VMEM((blk, npad), jnp.float32)],
        compiler_params=pltpu.CompilerParams(
            dimension_semantics=("arbitrary",)),
    )(emb, emb_pad)
    return out
